# Initial kernel scaffold; baseline (speedup 1.0000x reference)
import jax
import jax.numpy as jnp
from jax.experimental import pallas as pl


def kernel(x, scale, zero, choice_bits):
    raise NotImplementedError("write your pallas kernel here")


# speedup vs baseline: 1.8145x; 1.8145x over previous
import functools

import jax
import jax.numpy as jnp
from jax import lax
from jax.experimental import pallas as pl
from jax.experimental.pallas import tpu as pltpu
from jax.experimental.pallas import tpu_sc as plsc

NC = 2
NS = 16
L = 16
NW = NC * NS

MAXQ = 31
LUT_SIZE = MAXQ + 1


def _build_sc_call(N, K, R, unroll):
    rows_per_w = N // NW
    nblk = rows_per_w // R
    mesh = plsc.VectorSubcoreMesh(
        core_axis_name="c", subcore_axis_name="s", num_cores=NC,
        num_subcores=NS)

    @functools.partial(
        pl.kernel,
        out_type=jax.ShapeDtypeStruct((N, K), jnp.float32),
        mesh=mesh,
        compiler_params=pltpu.CompilerParams(needs_layout_passes=False),
        scratch_types=dict(
            lut_v=pltpu.VMEM((LUT_SIZE,), jnp.float32),
            sc_v=pltpu.VMEM((rows_per_w,), jnp.float32),
            rs_v=pltpu.VMEM((rows_per_w,), jnp.float32),
            zr_v=pltpu.VMEM((rows_per_w,), jnp.float32),
            inb=[pltpu.VMEM((R, K), jnp.float32) for _ in range(2)],
            outb=[pltpu.VMEM((R, K), jnp.float32) for _ in range(2)],
            insem=[pltpu.SemaphoreType.DMA for _ in range(2)],
            outsem=[pltpu.SemaphoreType.DMA for _ in range(2)],
        ),
    )
    def sc_quant(x_hbm, scale_hbm, rscale_hbm, zero_hbm, lut_hbm, out_hbm,
                 *, lut_v, sc_v, rs_v, zr_v, inb, outb, insem, outsem):
        wid = lax.axis_index("s") * NC + lax.axis_index("c")
        base_row = wid * rows_per_w

        pltpu.sync_copy(scale_hbm.at[pl.ds(base_row, rows_per_w)], sc_v)
        pltpu.sync_copy(rscale_hbm.at[pl.ds(base_row, rows_per_w)], rs_v)
        pltpu.sync_copy(zero_hbm.at[pl.ds(base_row, rows_per_w)], zr_v)
        pltpu.sync_copy(lut_hbm, lut_v)

        def bcast(ref, i):
            idx = jnp.full((L,), i, dtype=jnp.int32)
            return plsc.load_gather(ref, [idx])

        def in_start(blk, s):
            pltpu.async_copy(
                x_hbm.at[pl.ds(base_row + blk * R, R)], inb[s], insem[s])

        def in_wait(s):
            pltpu.make_async_copy(
                x_hbm.at[pl.ds(0, R)], inb[s], insem[s]).wait()

        def out_start(blk, s):
            pltpu.async_copy(
                outb[s], out_hbm.at[pl.ds(base_row + blk * R, R)],
                outsem[s])

        def out_wait(s):
            pltpu.make_async_copy(
                outb[s], out_hbm.at[pl.ds(0, R)], outsem[s]).wait()

        in_start(0, 0)
        in_start(1, 1)

        @pl.loop(0, nblk, step=2)
        def _pair(bp):
            for s in range(2):
                blk = bp + s
                in_wait(s)

                @pl.when(bp >= 2)
                def _():
                    out_wait(s)

                for r in range(R):
                    row_local = blk * R + r
                    sv = bcast(sc_v, row_local)
                    zv = bcast(zr_v, row_local)
                    rsv = bcast(rs_v, row_local)

                    @pl.loop(0, K // L, unroll=unroll)
                    def _chunk(c):
                        xv = inb[s][r, pl.ds(c * L, L)]
                        t = (xv - zv) * rsv
                        t = jnp.minimum(jnp.maximum(t, 0.0), float(MAXQ))
                        qi = (t + 0.5).astype(jnp.int32)
                        nl = plsc.load_gather(lut_v, [qi])
                        outb[s][r, pl.ds(c * L, L)] = nl * sv + zv

                out_start(blk, s)

                @pl.when(blk + 2 < nblk)
                def _():
                    in_start(blk + 2, s)

        out_wait(0)
        out_wait(1)

    return sc_quant


@functools.lru_cache(maxsize=None)
def _get_call(N, K):
    return jax.jit(_build_sc_call(N, K, R=4, unroll=8))


def kernel(x, scale, zero, choice_bits):
    scale = scale.astype(jnp.float32)
    rscale = 1.0 / scale
    grid = jnp.arange(LUT_SIZE, dtype=jnp.float32)
    dist = jnp.abs(grid[:, None] - choice_bits.astype(jnp.float32)[None, :])
    lut = jnp.take(choice_bits.astype(jnp.float32),
                   jnp.argmin(dist, axis=1), axis=0)
    call = _get_call(x.shape[0], x.shape[1])
    return call(x.astype(jnp.float32), scale, rscale,
                zero.astype(jnp.float32), lut)


# speedup vs baseline: 10.0062x; 5.5145x over previous
import functools

import jax
import jax.numpy as jnp
from jax import lax
from jax.experimental import pallas as pl
from jax.experimental.pallas import tpu as pltpu
from jax.experimental.pallas import tpu_sc as plsc

NC = 2
NS = 16
L = 16
NW = NC * NS

MAXQ = 31
LUT_SIZE = MAXQ + 1


def _build_sc_call(N, K, R, unroll):
    rows_per_w = N // NW
    nblk = rows_per_w // R
    mesh = plsc.VectorSubcoreMesh(
        core_axis_name="c", subcore_axis_name="s", num_cores=NC,
        num_subcores=NS)

    @functools.partial(
        pl.kernel,
        out_type=jax.ShapeDtypeStruct((N, K), jnp.float32),
        mesh=mesh,
        compiler_params=pltpu.CompilerParams(needs_layout_passes=False),
        scratch_types=dict(
            lut_v=pltpu.VMEM((LUT_SIZE,), jnp.float32),
            sc_v=pltpu.VMEM((rows_per_w,), jnp.float32),
            rs_v=pltpu.VMEM((rows_per_w,), jnp.float32),
            zr_v=pltpu.VMEM((rows_per_w,), jnp.float32),
            inb=[pltpu.VMEM((R, K), jnp.float32) for _ in range(2)],
            outb=[pltpu.VMEM((R, K), jnp.float32) for _ in range(2)],
            insem=[pltpu.SemaphoreType.DMA for _ in range(2)],
            outsem=[pltpu.SemaphoreType.DMA for _ in range(2)],
        ),
    )
    def sc_quant(x_hbm, scale_hbm, rscale_hbm, zero_hbm, lut_hbm, out_hbm,
                 *, lut_v, sc_v, rs_v, zr_v, inb, outb, insem, outsem):
        wid = lax.axis_index("s") * NC + lax.axis_index("c")
        base_row = wid * rows_per_w

        pltpu.sync_copy(scale_hbm.at[pl.ds(base_row, rows_per_w)], sc_v)
        pltpu.sync_copy(rscale_hbm.at[pl.ds(base_row, rows_per_w)], rs_v)
        pltpu.sync_copy(zero_hbm.at[pl.ds(base_row, rows_per_w)], zr_v)
        pltpu.sync_copy(lut_hbm, lut_v)

        def bcast(ref, i):
            idx = jnp.full((L,), i, dtype=jnp.int32)
            return plsc.load_gather(ref, [idx])

        def in_start(blk, s):
            pltpu.async_copy(
                x_hbm.at[pl.ds(base_row + blk * R, R)], inb[s], insem[s])

        def in_wait(s):
            pltpu.make_async_copy(
                x_hbm.at[pl.ds(0, R)], inb[s], insem[s]).wait()

        def out_start(blk, s):
            pltpu.async_copy(
                outb[s], out_hbm.at[pl.ds(base_row + blk * R, R)],
                outsem[s])

        def out_wait(s):
            pltpu.make_async_copy(
                outb[s], out_hbm.at[pl.ds(0, R)], outsem[s]).wait()

        in_start(0, 0)
        in_start(1, 1)

        @pl.loop(0, nblk, step=2)
        def _pair(bp):
            for s in range(2):
                blk = bp + s
                in_wait(s)

                @pl.when(bp >= 2)
                def _():
                    out_wait(s)

                for r in range(R):
                    row_local = blk * R + r
                    sv = bcast(sc_v, row_local)
                    zv = bcast(zr_v, row_local)
                    rsv = bcast(rs_v, row_local)

                    @plsc.parallel_loop(0, K // L, 1, unroll=unroll)
                    def _chunk(c):
                        xv = inb[s][r, pl.ds(c * L, L)]
                        t = (xv - zv) * rsv
                        t = jnp.minimum(jnp.maximum(t, 0.0), float(MAXQ))
                        qi = (t + 0.5).astype(jnp.int32)
                        nl = plsc.load_gather(lut_v, [qi])
                        outb[s][r, pl.ds(c * L, L)] = nl * sv + zv

                out_start(blk, s)

                @pl.when(blk + 2 < nblk)
                def _():
                    in_start(blk + 2, s)

        out_wait(0)
        out_wait(1)

    return sc_quant


@functools.lru_cache(maxsize=None)
def _get_call(N, K):
    return jax.jit(_build_sc_call(N, K, R=4, unroll=8))


def kernel(x, scale, zero, choice_bits):
    scale = scale.astype(jnp.float32)
    rscale = 1.0 / scale
    grid = jnp.arange(LUT_SIZE, dtype=jnp.float32)
    dist = jnp.abs(grid[:, None] - choice_bits.astype(jnp.float32)[None, :])
    lut = jnp.take(choice_bits.astype(jnp.float32),
                   jnp.argmin(dist, axis=1), axis=0)
    call = _get_call(x.shape[0], x.shape[1])
    return call(x.astype(jnp.float32), scale, rscale,
                zero.astype(jnp.float32), lut)


# speedup vs baseline: 10.4626x; 1.0456x over previous
import functools

import jax
import jax.numpy as jnp
from jax import lax
from jax.experimental import pallas as pl
from jax.experimental.pallas import tpu as pltpu
from jax.experimental.pallas import tpu_sc as plsc

NC = 2
NS = 16
L = 16
NW = NC * NS

MAXQ = 31
LUT_SIZE = MAXQ + 1


def _build_sc_call(N, K, R, unroll):
    rows_per_w = N // NW
    nblk = rows_per_w // R
    mesh = plsc.VectorSubcoreMesh(
        core_axis_name="c", subcore_axis_name="s", num_cores=NC,
        num_subcores=NS)

    @functools.partial(
        pl.kernel,
        out_type=jax.ShapeDtypeStruct((N, K), jnp.float32),
        mesh=mesh,
        compiler_params=pltpu.CompilerParams(needs_layout_passes=False),
        scratch_types=dict(
            lut_v=pltpu.VMEM((LUT_SIZE,), jnp.float32),
            sc_v=pltpu.VMEM((rows_per_w,), jnp.float32),
            rs_v=pltpu.VMEM((rows_per_w,), jnp.float32),
            zr_v=pltpu.VMEM((rows_per_w,), jnp.float32),
            inb=[pltpu.VMEM((R, K), jnp.float32) for _ in range(2)],
            outb=[pltpu.VMEM((R, K), jnp.float32) for _ in range(2)],
            insem=[pltpu.SemaphoreType.DMA for _ in range(2)],
            outsem=[pltpu.SemaphoreType.DMA for _ in range(2)],
        ),
    )
    def sc_quant(x_hbm, scale_hbm, rscale_hbm, zero_hbm, lut_hbm, out_hbm,
                 *, lut_v, sc_v, rs_v, zr_v, inb, outb, insem, outsem):
        wid = lax.axis_index("s") * NC + lax.axis_index("c")
        base_row = wid * rows_per_w

        pltpu.sync_copy(scale_hbm.at[pl.ds(base_row, rows_per_w)], sc_v)
        pltpu.sync_copy(rscale_hbm.at[pl.ds(base_row, rows_per_w)], rs_v)
        pltpu.sync_copy(zero_hbm.at[pl.ds(base_row, rows_per_w)], zr_v)
        pltpu.sync_copy(lut_hbm, lut_v)

        def bcast(ref, i):
            idx = jnp.full((L,), i, dtype=jnp.int32)
            return plsc.load_gather(ref, [idx])

        def in_start(blk, s):
            pltpu.async_copy(
                x_hbm.at[pl.ds(base_row + blk * R, R)], inb[s], insem[s])

        def in_wait(s):
            pltpu.make_async_copy(
                x_hbm.at[pl.ds(0, R)], inb[s], insem[s]).wait()

        def out_start(blk, s):
            pltpu.async_copy(
                outb[s], out_hbm.at[pl.ds(base_row + blk * R, R)],
                outsem[s])

        def out_wait(s):
            pltpu.make_async_copy(
                outb[s], out_hbm.at[pl.ds(0, R)], outsem[s]).wait()

        in_start(0, 0)
        in_start(1, 1)

        @pl.loop(0, nblk, step=2)
        def _pair(bp):
            for s in range(2):
                blk = bp + s
                in_wait(s)

                @pl.when(bp >= 2)
                def _():
                    out_wait(s)

                for r in range(R):
                    row_local = blk * R + r
                    sv = bcast(sc_v, row_local)
                    zv = bcast(zr_v, row_local)
                    rsv = bcast(rs_v, row_local)
                    cv = 0.5 - zv * rsv

                    @plsc.parallel_loop(0, K // L, 1, unroll=unroll)
                    def _chunk(c):
                        xv = inb[s][r, pl.ds(c * L, L)]
                        t = xv * rsv + cv
                        t = jnp.minimum(jnp.maximum(t, 0.5), MAXQ + 0.5)
                        qi = t.astype(jnp.int32)
                        nl = plsc.load_gather(lut_v, [qi])
                        outb[s][r, pl.ds(c * L, L)] = nl * sv + zv

                out_start(blk, s)

                @pl.when(blk + 2 < nblk)
                def _():
                    in_start(blk + 2, s)

        out_wait(0)
        out_wait(1)

    return sc_quant


@functools.lru_cache(maxsize=None)
def _get_call(N, K):
    return jax.jit(_build_sc_call(N, K, R=4, unroll=8))


def kernel(x, scale, zero, choice_bits):
    scale = scale.astype(jnp.float32)
    rscale = 1.0 / scale
    grid = jnp.arange(LUT_SIZE, dtype=jnp.float32)
    dist = jnp.abs(grid[:, None] - choice_bits.astype(jnp.float32)[None, :])
    lut = jnp.take(choice_bits.astype(jnp.float32),
                   jnp.argmin(dist, axis=1), axis=0)
    call = _get_call(x.shape[0], x.shape[1])
    return call(x.astype(jnp.float32), scale, rscale,
                zero.astype(jnp.float32), lut)
